# baseline (speedup 1.0000x reference)
import functools

import jax
import jax.numpy as jnp
from jax import lax
from jax.experimental import pallas as pl
from jax.experimental.pallas import tpu as pltpu
from jax.experimental.pallas import tpu_sc as plsc

HIDDEN = 128
NC = 2
NS = 16
NW = NC * NS

LANE = 128
SLOT = 2
K = 3


def _make_gather(n_idx_rows):
    rows_per_w = n_idx_rows // NW
    n_slots = rows_per_w // SLOT
    n_groups = (n_slots - 1) // K
    assert n_groups * K == n_slots - 1
    mesh = plsc.VectorSubcoreMesh(core_axis_name="c", subcore_axis_name="s")

    @functools.partial(
        pl.kernel,
        mesh=mesh,
        out_type=jax.ShapeDtypeStruct((n_idx_rows * LANE, HIDDEN), jnp.float32),
        scratch_types=(
            [pltpu.VMEM((SLOT, LANE), jnp.int32)] * K
            + [pltpu.VMEM((SLOT * LANE, HIDDEN), jnp.float32)] * K
            + [pltpu.SemaphoreType.DMA] * (3 * K)
        ),
    )
    def gather_kernel(idx_hbm, table_hbm, out_hbm, *refs):
        wid = lax.axis_index("s") * NC + lax.axis_index("c")
        w_row0 = wid * rows_per_w
        IV = refs[0:K]
        RV = refs[K:2 * K]
        GS = refs[2 * K:3 * K]
        OS = refs[3 * K:4 * K]
        IS = refs[4 * K:5 * K]

        def drain_out(b):
            pltpu.make_async_copy(RV[b], out_hbm.at[pl.ds(0, SLOT * LANE)],
                                  OS[b]).wait()

        def prefetch_idx(b, row0):
            pltpu.async_copy(idx_hbm.at[pl.ds(row0, SLOT)], IV[b], IS[b])

        def fire_gathers(b):
            pltpu.make_async_copy(idx_hbm.at[pl.ds(0, SLOT)], IV[b],
                                  IS[b]).wait()
            for j in range(SLOT):
                pltpu.async_copy(table_hbm.at[IV[b].at[j]],
                                 RV[b].at[pl.ds(j * LANE, LANE)], GS[b])

        def wait_gathers(b):
            for j in range(SLOT):
                pltpu.make_async_copy(table_hbm.at[IV[b].at[j]],
                                      RV[b].at[pl.ds(j * LANE, LANE)],
                                      GS[b]).wait()

        def fire_out(b, t):
            pltpu.async_copy(
                RV[b],
                out_hbm.at[pl.ds((w_row0 + t * SLOT) * LANE, SLOT * LANE)],
                OS[b])

        for b in range(K):
            prefetch_idx(b, w_row0 + b * SLOT)
        fire_gathers(0)

        def group(q, carry):
            for b in range(K):
                t = K * q + b
                bf = (b + 1) % K
                if b < 2:
                    pl.when(q >= 1)(lambda bf=bf: drain_out(bf))
                else:
                    drain_out(bf)
                fire_gathers(bf)
                wait_gathers(b)
                if b == 0:
                    prefetch_idx(b, w_row0 + (t + K) * SLOT)
                else:
                    pl.when(q < n_groups - 1)(
                        lambda b=b, t=t:
                        prefetch_idx(b, w_row0 + (t + K) * SLOT))
                fire_out(b, t)
            return carry

        lax.fori_loop(0, n_groups, group, 0)
        t_last = n_slots - 1
        drain_out(1)
        wait_gathers(0)
        fire_out(0, t_last)
        drain_out(2)
        drain_out(0)

    return gather_kernel


def kernel(batch_token_ids, token_embedding):
    b, s = batch_token_ids.shape
    n = b * s
    idx2d = batch_token_ids.reshape(n // LANE, LANE).astype(jnp.int32)
    out = _make_gather(n // LANE)(idx2d, token_embedding)
    return out.reshape(b, s, HIDDEN)
